# baseline (speedup 1.0000x reference)
import numpy as np
import jax
import jax.numpy as jnp
from jax.experimental import pallas as pl

R_CUTOFF = 5.0
NUM_RBF = 6
NUM_SBF = 7
ENVELOPE_P = 6


def _sph_jn_np(l, x):
    x = np.asarray(x, dtype=np.float64)
    j0 = np.sin(x) / x
    if l == 0:
        return j0
    j1 = np.sin(x) / x**2 - np.cos(x) / x
    if l == 1:
        return j1
    jm, jc = j0, j1
    for ll in range(1, l):
        jm, jc = jc, (2 * ll + 1) / x * jc - jm
    return jc


def _bessel_roots(num_l, num_n):
    roots = np.zeros((num_l, num_n))
    xs = np.linspace(1e-3, 60.0, 120000)
    for l in range(num_l):
        vals = _sph_jn_np(l, xs)
        sgn = np.sign(vals)
        cross = np.where(sgn[:-1] * sgn[1:] < 0)[0][:num_n]
        for n, i in enumerate(cross):
            a, b = xs[i], xs[i + 1]
            fa = _sph_jn_np(l, np.array([a]))[0]
            for _ in range(50):
                m = 0.5 * (a + b)
                fm = _sph_jn_np(l, np.array([m]))[0]
                if fa * fm <= 0:
                    b = m
                else:
                    a, fa = m, fm
            roots[l, n] = 0.5 * (a + b)
    return roots


_ROOTS = _bessel_roots(NUM_SBF, NUM_RBF)
_SBF_NORM = np.zeros_like(_ROOTS)
for _l in range(NUM_SBF):
    for _n in range(NUM_RBF):
        _SBF_NORM[_l, _n] = np.sqrt(
            2.0 / (R_CUTOFF**3 * _sph_jn_np(_l + 1, np.array([_ROOTS[_l, _n]]))[0] ** 2))
_YLM = np.sqrt((2 * np.arange(NUM_SBF) + 1) / (4 * np.pi))


def _mm(a, b):
    return jnp.dot(a, b, preferred_element_type=jnp.float32)


def _swish(x):
    return x * jax.nn.sigmoid(x)


def _envelope(x):
    p = ENVELOPE_P
    a = -(p + 1) * (p + 2) / 2.0
    b = p * (p + 2)
    c = -p * (p + 1) / 2.0
    env = 1.0 + a * x**p + b * x**(p + 1) + c * x**(p + 2)
    return jnp.where(x < 1.0, env, 0.0)


def _sph_jn(l, x):
    j0 = jnp.sin(x) / x
    if l == 0:
        return j0
    j1 = jnp.sin(x) / (x * x) - jnp.cos(x) / x
    if l == 1:
        return j1
    jm, jc = j0, j1
    for ll in range(1, l):
        jm, jc = jc, (2 * ll + 1) / x * jc - jm
    return jc


def _edge1_body(d_ref, sj_ref, si_ref, freqs_ref, A_ref, B_ref, Wc_ref, bemb_ref,
                Wrbf0_ref, Wji_ref, bji_ref, Wkj_ref, bkj_ref, Wr1_ref, Wr2_ref,
                Wdown_ref, m_ref, t0_ref, xji_ref, xkjd_ref, rbf_ref):
    d = d_ref[:, :]
    x = d / R_CUTOFF
    env = _envelope(x)
    freqs = freqs_ref[:, :]
    rbf = np.float32(np.sqrt(2.0 / R_CUTOFF)) * jnp.sin(x * freqs) / d * env
    T = d.shape[0]
    iota8 = jax.lax.broadcasted_iota(jnp.int32, (T, 8), 1)
    ohj = (iota8 == sj_ref[:, :]).astype(jnp.float32)
    ohi = (iota8 == si_ref[:, :]).astype(jnp.float32)
    hp = jax.lax.Precision.HIGHEST
    pre = (jnp.dot(ohj, A_ref[:, :], precision=hp,
                   preferred_element_type=jnp.float32)
           + jnp.dot(ohi, B_ref[:, :], precision=hp,
                     preferred_element_type=jnp.float32)
           + _mm(rbf, Wc_ref[:, :]) + bemb_ref[:, :])
    m = _swish(pre)
    x_ji = _swish(_mm(m, Wji_ref[:, :]) + bji_ref[:, :])
    x_kj = _swish(_mm(m, Wkj_ref[:, :]) + bkj_ref[:, :])
    rbf_e = _mm(_mm(rbf, Wr1_ref[:, :]), Wr2_ref[:, :])
    xkjd = _swish(_mm(x_kj * rbf_e, Wdown_ref[:, :]))
    m_ref[:, :] = m
    t0_ref[:, :] = _mm(rbf, Wrbf0_ref[:, :]) * m
    xji_ref[:, :] = x_ji
    xkjd_ref[:, :] = xkjd
    rbf_ref[:, :] = rbf


def _tri_body(ang_ref, dkj_ref, mask_ref, g_ref, roots_ref, norm_ref,
              Ws1_ref, Ws2_ref, o_ref):
    x = dkj_ref[:, :] / R_CUTOFF
    env = _envelope(x)
    cos_t = jnp.cos(ang_ref[:, :])
    roots = roots_ref[:, :]
    norms = norm_ref[:, :]
    Ps = [jnp.ones_like(cos_t), cos_t]
    for l in range(2, NUM_SBF):
        Ps.append(((2 * l - 1) * cos_t * Ps[-1] - (l - 1) * Ps[-2]) / l)
    parts = []
    for l in range(NUM_SBF):
        root = roots[l:l + 1, :NUM_RBF]
        norm = norms[l:l + 1, :NUM_RBF]
        jl = _sph_jn(l, x * root)
        parts.append(jl * norm * (Ps[l] * np.float32(_YLM[l])))
    sbf = jnp.concatenate(parts, axis=1) * env * mask_ref[:, :]
    sbfe = _mm(_mm(sbf, Ws1_ref[:, :]), Ws2_ref[:, :])
    o_ref[:, :] = g_ref[:, :] * sbfe


def _edge2_body(seg_ref, xji_ref, m_ref, rbf_ref, Wup_ref,
                rbW1_ref, rbb1_ref, rbW2_ref, rbb2_ref,
                Wskip_ref, bskip_ref,
                raW1a_ref, rab1a_ref, raW2a_ref, rab2a_ref,
                raW1b_ref, rab1b_ref, raW2b_ref, rab2b_ref,
                Wrbf1_ref, t1_ref):
    x_kj = _swish(_mm(seg_ref[:, :], Wup_ref[:, :]))
    h = xji_ref[:, :] + x_kj
    h = h + _swish(_mm(_swish(_mm(h, rbW1_ref[:, :]) + rbb1_ref[:, :]),
                       rbW2_ref[:, :]) + rbb2_ref[:, :])
    h = _swish(_mm(h, Wskip_ref[:, :]) + bskip_ref[:, :]) + m_ref[:, :]
    h = h + _swish(_mm(_swish(_mm(h, raW1a_ref[:, :]) + rab1a_ref[:, :]),
                       raW2a_ref[:, :]) + rab2a_ref[:, :])
    h = h + _swish(_mm(_swish(_mm(h, raW1b_ref[:, :]) + rab1b_ref[:, :]),
                       raW2b_ref[:, :]) + rab2b_ref[:, :])
    t1_ref[:, :] = _mm(rbf_ref[:, :], Wrbf1_ref[:, :]) * h


def _scatter_body(idx_ref, t0_ref, t1_ref, pa0_ref, pa1_ref, *, ablock):
    a = pl.program_id(0)
    e = pl.program_id(1)
    idx = idx_ref[:, :]
    Te = idx.shape[0]
    iota = jax.lax.broadcasted_iota(jnp.int32, (Te, ablock), 1) + a * ablock
    oh = (iota == idx).astype(jnp.float32)
    dn = (((0,), (0,)), ((), ()))
    hp = jax.lax.Precision.HIGHEST
    c0 = jax.lax.dot_general(oh, t0_ref[:, :], dn, precision=hp,
                             preferred_element_type=jnp.float32)
    c1 = jax.lax.dot_general(oh, t1_ref[:, :], dn, precision=hp,
                             preferred_element_type=jnp.float32)

    @pl.when(e == 0)
    def _init():
        pa0_ref[:, :] = c0
        pa1_ref[:, :] = c1

    @pl.when(e != 0)
    def _acc():
        pa0_ref[:, :] += c0
        pa1_ref[:, :] += c1


def _atom_body(pa0_ref, pa1_ref,
               Wu0_ref, D01W_ref, D01b_ref, D02W_ref, D02b_ref, D03W_ref,
               D03b_ref, Wf0_ref,
               Wu1_ref, D11W_ref, D11b_ref, D12W_ref, D12b_ref, D13W_ref,
               D13b_ref, Wf1_ref, o_ref):
    y0 = _mm(pa0_ref[:, :], Wu0_ref[:, :])
    y0 = _swish(_mm(y0, D01W_ref[:, :]) + D01b_ref[:, :])
    y0 = _swish(_mm(y0, D02W_ref[:, :]) + D02b_ref[:, :])
    y0 = _swish(_mm(y0, D03W_ref[:, :]) + D03b_ref[:, :])
    y0 = _mm(y0, Wf0_ref[:, :])
    y1 = _mm(pa1_ref[:, :], Wu1_ref[:, :])
    y1 = _swish(_mm(y1, D11W_ref[:, :]) + D11b_ref[:, :])
    y1 = _swish(_mm(y1, D12W_ref[:, :]) + D12b_ref[:, :])
    y1 = _swish(_mm(y1, D13W_ref[:, :]) + D13b_ref[:, :])
    y1 = _mm(y1, Wf1_ref[:, :])
    o_ref[:, :] = y0 + y1


def _full(arr):
    return pl.BlockSpec(arr.shape, lambda *_: tuple(0 for _ in arr.shape))


def kernel(params, species, distance_ij, angles, edge_mask, triplet_mask,
           idx_i, idx_j, expand_to_kj, reduce_to_ji):
    f32 = jnp.float32
    E = distance_ij.shape[0]
    Nt = angles.shape[0]
    N = species.shape[0]
    ib = params['int_blocks'][0]
    ob0 = params['out_blocks'][0]
    ob1 = params['out_blocks'][1]

    d = jnp.where(edge_mask, distance_ij, 2.0 * R_CUTOFF)
    d2 = d.reshape(E, 1)
    sj = species[idx_j].reshape(E, 1)
    si = species[idx_i].reshape(E, 1)

    TE = params['type_emb'].shape[1]
    emb_W = params['emb_W']
    A8 = params['type_emb'] @ emb_W[:TE]
    B8 = params['type_emb'] @ emb_W[TE:2 * TE]
    Wc = emb_W[2 * TE:]
    freqs = params['rbf_freqs'].reshape(1, -1)
    r1 = lambda v: v.reshape(1, -1)

    Te = 2000
    ge = E // Te
    es = lambda w: pl.BlockSpec((Te, w), lambda e: (e, 0))
    EMB = emb_W.shape[1]
    DOWN = ib['W_down'].shape[1]

    edge1_in = [d2, sj, si, freqs, A8, B8, Wc, r1(params['emb_b']),
                ob0['W_rbf'], ib['W_ji'], r1(ib['b_ji']), ib['W_kj'],
                r1(ib['b_kj']), ib['W_rbf1'], ib['W_rbf2'], ib['W_down']]
    m, t0, x_ji, xkjd, rbf = pl.pallas_call(
        _edge1_body,
        grid=(ge,),
        in_specs=[es(1), es(1), es(1)] + [_full(a) for a in edge1_in[3:]],
        out_specs=[es(EMB), es(EMB), es(EMB), es(DOWN), es(NUM_RBF)],
        out_shape=[jax.ShapeDtypeStruct((E, EMB), f32),
                   jax.ShapeDtypeStruct((E, EMB), f32),
                   jax.ShapeDtypeStruct((E, EMB), f32),
                   jax.ShapeDtypeStruct((E, DOWN), f32),
                   jax.ShapeDtypeStruct((E, NUM_RBF), f32)],
    )(*edge1_in)

    g = xkjd[expand_to_kj]
    dkj = d[expand_to_kj].reshape(Nt, 1)
    ang2 = angles.reshape(Nt, 1)
    tm = triplet_mask.astype(f32).reshape(Nt, 1)

    Tt = 2000
    gt = Nt // Tt
    ts = lambda w: pl.BlockSpec((Tt, w), lambda e: (e, 0))
    rootsP = jnp.asarray(np.pad(_ROOTS, ((0, 1), (0, 2)),
                                constant_values=1.0), f32)
    normP = jnp.asarray(np.pad(_SBF_NORM, ((0, 1), (0, 2))), f32)
    tri_in = [ang2, dkj, tm, g, rootsP, normP, ib['W_sbf1'], ib['W_sbf2']]
    tri_out = pl.pallas_call(
        _tri_body,
        grid=(gt,),
        in_specs=[ts(1), ts(1), ts(1), ts(DOWN), _full(rootsP), _full(normP),
                  _full(ib['W_sbf1']), _full(ib['W_sbf2'])],
        out_specs=ts(ib['W_sbf2'].shape[1]),
        out_shape=jax.ShapeDtypeStruct((Nt, ib['W_sbf2'].shape[1]), f32),
    )(*tri_in)

    seg = jax.ops.segment_sum(tri_out, reduce_to_ji, num_segments=E)

    (rbW1, rbb1, rbW2, rbb2), = ib['res_before']
    (raW1a, rab1a, raW2a, rab2a), (raW1b, rab1b, raW2b, rab2b) = ib['res_after']
    edge2_in = [seg, x_ji, m, rbf, ib['W_up'],
                rbW1, r1(rbb1), rbW2, r1(rbb2),
                ib['W_skip'], r1(ib['b_skip']),
                raW1a, r1(rab1a), raW2a, r1(rab2a),
                raW1b, r1(rab1b), raW2b, r1(rab2b),
                ob1['W_rbf']]
    t1 = pl.pallas_call(
        _edge2_body,
        grid=(ge,),
        in_specs=[es(DOWN), es(EMB), es(EMB), es(NUM_RBF)]
        + [_full(a) for a in edge2_in[4:]],
        out_specs=es(EMB),
        out_shape=jax.ShapeDtypeStruct((E, EMB), f32),
    )(*edge2_in)

    AB = 1000
    ga = N // AB
    import functools
    pa0, pa1 = pl.pallas_call(
        functools.partial(_scatter_body, ablock=AB),
        grid=(ga, ge),
        in_specs=[pl.BlockSpec((Te, 1), lambda a, e: (e, 0)),
                  pl.BlockSpec((Te, EMB), lambda a, e: (e, 0)),
                  pl.BlockSpec((Te, EMB), lambda a, e: (e, 0))],
        out_specs=[pl.BlockSpec((AB, EMB), lambda a, e: (a, 0)),
                   pl.BlockSpec((AB, EMB), lambda a, e: (a, 0))],
        out_shape=[jax.ShapeDtypeStruct((N, EMB), f32),
                   jax.ShapeDtypeStruct((N, EMB), f32)],
    )(idx_i.reshape(E, 1), t0, t1)

    (d01W, d01b), (d02W, d02b), (d03W, d03b) = ob0['dense']
    (d11W, d11b), (d12W, d12b), (d13W, d13b) = ob1['dense']
    atom_in = [pa0, pa1,
               ob0['W_up'], d01W, r1(d01b), d02W, r1(d02b), d03W, r1(d03b),
               ob0['W_final'],
               ob1['W_up'], d11W, r1(d11b), d12W, r1(d12b), d13W, r1(d13b),
               ob1['W_final']]
    as_ = lambda w: pl.BlockSpec((AB, w), lambda a: (a, 0))
    out = pl.pallas_call(
        _atom_body,
        grid=(ga,),
        in_specs=[as_(EMB), as_(EMB)] + [_full(a) for a in atom_in[2:]],
        out_specs=as_(ob0['W_final'].shape[1]),
        out_shape=jax.ShapeDtypeStruct((N, ob0['W_final'].shape[1]), f32),
    )(*atom_in)
    return out


# speedup vs baseline: 1.2063x; 1.2063x over previous
import numpy as np
import jax
import jax.numpy as jnp
from jax.experimental import pallas as pl

R_CUTOFF = 5.0
NUM_RBF = 6
NUM_SBF = 7
ENVELOPE_P = 6


def _sph_jn_np(l, x):
    x = np.asarray(x, dtype=np.float64)
    j0 = np.sin(x) / x
    if l == 0:
        return j0
    j1 = np.sin(x) / x**2 - np.cos(x) / x
    if l == 1:
        return j1
    jm, jc = j0, j1
    for ll in range(1, l):
        jm, jc = jc, (2 * ll + 1) / x * jc - jm
    return jc


def _bessel_roots(num_l, num_n):
    roots = np.zeros((num_l, num_n))
    xs = np.linspace(1e-3, 60.0, 120000)
    for l in range(num_l):
        vals = _sph_jn_np(l, xs)
        sgn = np.sign(vals)
        cross = np.where(sgn[:-1] * sgn[1:] < 0)[0][:num_n]
        for n, i in enumerate(cross):
            a, b = xs[i], xs[i + 1]
            fa = _sph_jn_np(l, np.array([a]))[0]
            for _ in range(50):
                m = 0.5 * (a + b)
                fm = _sph_jn_np(l, np.array([m]))[0]
                if fa * fm <= 0:
                    b = m
                else:
                    a, fa = m, fm
            roots[l, n] = 0.5 * (a + b)
    return roots


_ROOTS = _bessel_roots(NUM_SBF, NUM_RBF)
_SBF_NORM = np.zeros_like(_ROOTS)
for _l in range(NUM_SBF):
    for _n in range(NUM_RBF):
        _SBF_NORM[_l, _n] = np.sqrt(
            2.0 / (R_CUTOFF**3 * _sph_jn_np(_l + 1, np.array([_ROOTS[_l, _n]]))[0] ** 2))
_YLM = np.sqrt((2 * np.arange(NUM_SBF) + 1) / (4 * np.pi))


def _mm(a, b):
    return jnp.dot(a, b, preferred_element_type=jnp.float32)


def _swish(x):
    return x * jax.nn.sigmoid(x)


def _envelope(x):
    p = ENVELOPE_P
    a = -(p + 1) * (p + 2) / 2.0
    b = p * (p + 2)
    c = -p * (p + 1) / 2.0
    env = 1.0 + a * x**p + b * x**(p + 1) + c * x**(p + 2)
    return jnp.where(x < 1.0, env, 0.0)


def _sph_jn(l, x):
    j0 = jnp.sin(x) / x
    if l == 0:
        return j0
    j1 = jnp.sin(x) / (x * x) - jnp.cos(x) / x
    if l == 1:
        return j1
    jm, jc = j0, j1
    for ll in range(1, l):
        jm, jc = jc, (2 * ll + 1) / x * jc - jm
    return jc


def _edge1_body(d_ref, sj_ref, si_ref, freqs_ref, A_ref, B_ref, Wc_ref, bemb_ref,
                Wrbf0_ref, Wji_ref, bji_ref, Wkj_ref, bkj_ref, Wr1_ref, Wr2_ref,
                Wdown_ref, m_ref, t0_ref, xji_ref, xkjd_ref, rbf_ref):
    d = d_ref[:, :]
    x = d / R_CUTOFF
    env = _envelope(x)
    freqs = freqs_ref[:, :]
    rbf = np.float32(np.sqrt(2.0 / R_CUTOFF)) * jnp.sin(x * freqs) / d * env
    T = d.shape[0]
    iota8 = jax.lax.broadcasted_iota(jnp.int32, (T, 8), 1)
    ohj = (iota8 == sj_ref[:, :]).astype(jnp.float32)
    ohi = (iota8 == si_ref[:, :]).astype(jnp.float32)
    hp = jax.lax.Precision.HIGHEST
    pre = (jnp.dot(ohj, A_ref[:, :], precision=hp,
                   preferred_element_type=jnp.float32)
           + jnp.dot(ohi, B_ref[:, :], precision=hp,
                     preferred_element_type=jnp.float32)
           + _mm(rbf, Wc_ref[:, :]) + bemb_ref[:, :])
    m = _swish(pre)
    x_ji = _swish(_mm(m, Wji_ref[:, :]) + bji_ref[:, :])
    x_kj = _swish(_mm(m, Wkj_ref[:, :]) + bkj_ref[:, :])
    rbf_e = _mm(_mm(rbf, Wr1_ref[:, :]), Wr2_ref[:, :])
    xkjd = _swish(_mm(x_kj * rbf_e, Wdown_ref[:, :]))
    m_ref[:, :] = m
    t0_ref[:, :] = _mm(rbf, Wrbf0_ref[:, :]) * m
    xji_ref[:, :] = x_ji
    xkjd_ref[:, :] = xkjd
    rbf_ref[:, :] = rbf


def _tri_body(ang_ref, dkj_ref, mask_ref, g_ref, roots_ref, norm_ref,
              Ws1_ref, Ws2_ref, o_ref):
    x = dkj_ref[:, :] / R_CUTOFF
    env = _envelope(x)
    cos_t = jnp.cos(ang_ref[:, :])
    roots = roots_ref[:, :]
    norms = norm_ref[:, :]
    Ps = [jnp.ones_like(cos_t), cos_t]
    for l in range(2, NUM_SBF):
        Ps.append(((2 * l - 1) * cos_t * Ps[-1] - (l - 1) * Ps[-2]) / l)
    parts = []
    for l in range(NUM_SBF):
        root = roots[l:l + 1, :NUM_RBF]
        norm = norms[l:l + 1, :NUM_RBF]
        jl = _sph_jn(l, x * root)
        parts.append(jl * norm * (Ps[l] * np.float32(_YLM[l])))
    sbf = jnp.concatenate(parts, axis=1) * env * mask_ref[:, :]
    sbfe = _mm(_mm(sbf, Ws1_ref[:, :]), Ws2_ref[:, :])
    o_ref[:, :] = g_ref[:, :] * sbfe


def _edge2_body(seg_ref, xji_ref, m_ref, rbf_ref, Wup_ref,
                rbW1_ref, rbb1_ref, rbW2_ref, rbb2_ref,
                Wskip_ref, bskip_ref,
                raW1a_ref, rab1a_ref, raW2a_ref, rab2a_ref,
                raW1b_ref, rab1b_ref, raW2b_ref, rab2b_ref,
                Wrbf1_ref, t1_ref):
    x_kj = _swish(_mm(seg_ref[:, :], Wup_ref[:, :]))
    h = xji_ref[:, :] + x_kj
    h = h + _swish(_mm(_swish(_mm(h, rbW1_ref[:, :]) + rbb1_ref[:, :]),
                       rbW2_ref[:, :]) + rbb2_ref[:, :])
    h = _swish(_mm(h, Wskip_ref[:, :]) + bskip_ref[:, :]) + m_ref[:, :]
    h = h + _swish(_mm(_swish(_mm(h, raW1a_ref[:, :]) + rab1a_ref[:, :]),
                       raW2a_ref[:, :]) + rab2a_ref[:, :])
    h = h + _swish(_mm(_swish(_mm(h, raW1b_ref[:, :]) + rab1b_ref[:, :]),
                       raW2b_ref[:, :]) + rab2b_ref[:, :])
    t1_ref[:, :] = _mm(rbf_ref[:, :], Wrbf1_ref[:, :]) * h


def _scatter_body(idx_ref, t0_ref, t1_ref, pa0_ref, pa1_ref, *, ablock):
    a = pl.program_id(0)
    e = pl.program_id(1)
    idx = idx_ref[:, :]
    Te = idx.shape[0]
    iota = jax.lax.broadcasted_iota(jnp.int32, (Te, ablock), 1) + a * ablock
    oh = (iota == idx).astype(jnp.float32)
    dn = (((0,), (0,)), ((), ()))

    def _split_dot(t):
        hi = t.astype(jnp.bfloat16).astype(jnp.float32)
        lo = t - hi
        return (jax.lax.dot_general(oh, hi, dn,
                                    preferred_element_type=jnp.float32)
                + jax.lax.dot_general(oh, lo, dn,
                                      preferred_element_type=jnp.float32))

    c0 = _split_dot(t0_ref[:, :])
    c1 = _split_dot(t1_ref[:, :])

    @pl.when(e == 0)
    def _init():
        pa0_ref[:, :] = c0
        pa1_ref[:, :] = c1

    @pl.when(e != 0)
    def _acc():
        pa0_ref[:, :] += c0
        pa1_ref[:, :] += c1


def _atom_body(pa0_ref, pa1_ref,
               Wu0_ref, D01W_ref, D01b_ref, D02W_ref, D02b_ref, D03W_ref,
               D03b_ref, Wf0_ref,
               Wu1_ref, D11W_ref, D11b_ref, D12W_ref, D12b_ref, D13W_ref,
               D13b_ref, Wf1_ref, o_ref):
    y0 = _mm(pa0_ref[:, :], Wu0_ref[:, :])
    y0 = _swish(_mm(y0, D01W_ref[:, :]) + D01b_ref[:, :])
    y0 = _swish(_mm(y0, D02W_ref[:, :]) + D02b_ref[:, :])
    y0 = _swish(_mm(y0, D03W_ref[:, :]) + D03b_ref[:, :])
    y0 = _mm(y0, Wf0_ref[:, :])
    y1 = _mm(pa1_ref[:, :], Wu1_ref[:, :])
    y1 = _swish(_mm(y1, D11W_ref[:, :]) + D11b_ref[:, :])
    y1 = _swish(_mm(y1, D12W_ref[:, :]) + D12b_ref[:, :])
    y1 = _swish(_mm(y1, D13W_ref[:, :]) + D13b_ref[:, :])
    y1 = _mm(y1, Wf1_ref[:, :])
    o_ref[:, :] = y0 + y1


def _full(arr):
    return pl.BlockSpec(arr.shape, lambda *_: tuple(0 for _ in arr.shape))


def kernel(params, species, distance_ij, angles, edge_mask, triplet_mask,
           idx_i, idx_j, expand_to_kj, reduce_to_ji):
    f32 = jnp.float32
    E = distance_ij.shape[0]
    Nt = angles.shape[0]
    N = species.shape[0]
    ib = params['int_blocks'][0]
    ob0 = params['out_blocks'][0]
    ob1 = params['out_blocks'][1]

    d = jnp.where(edge_mask, distance_ij, 2.0 * R_CUTOFF)
    d2 = d.reshape(E, 1)
    sj = species[idx_j].reshape(E, 1)
    si = species[idx_i].reshape(E, 1)

    TE = params['type_emb'].shape[1]
    emb_W = params['emb_W']
    A8 = params['type_emb'] @ emb_W[:TE]
    B8 = params['type_emb'] @ emb_W[TE:2 * TE]
    Wc = emb_W[2 * TE:]
    freqs = params['rbf_freqs'].reshape(1, -1)
    r1 = lambda v: v.reshape(1, -1)

    Te = 2000
    ge = E // Te
    es = lambda w: pl.BlockSpec((Te, w), lambda e: (e, 0))
    EMB = emb_W.shape[1]
    DOWN = ib['W_down'].shape[1]

    edge1_in = [d2, sj, si, freqs, A8, B8, Wc, r1(params['emb_b']),
                ob0['W_rbf'], ib['W_ji'], r1(ib['b_ji']), ib['W_kj'],
                r1(ib['b_kj']), ib['W_rbf1'], ib['W_rbf2'], ib['W_down']]
    m, t0, x_ji, xkjd, rbf = pl.pallas_call(
        _edge1_body,
        grid=(ge,),
        in_specs=[es(1), es(1), es(1)] + [_full(a) for a in edge1_in[3:]],
        out_specs=[es(EMB), es(EMB), es(EMB), es(DOWN), es(NUM_RBF)],
        out_shape=[jax.ShapeDtypeStruct((E, EMB), f32),
                   jax.ShapeDtypeStruct((E, EMB), f32),
                   jax.ShapeDtypeStruct((E, EMB), f32),
                   jax.ShapeDtypeStruct((E, DOWN), f32),
                   jax.ShapeDtypeStruct((E, NUM_RBF), f32)],
    )(*edge1_in)

    g = xkjd[expand_to_kj]
    dkj = d[expand_to_kj].reshape(Nt, 1)
    ang2 = angles.reshape(Nt, 1)
    tm = triplet_mask.astype(f32).reshape(Nt, 1)

    Tt = 4000
    gt = Nt // Tt
    ts = lambda w: pl.BlockSpec((Tt, w), lambda e: (e, 0))
    rootsP = jnp.asarray(np.pad(_ROOTS, ((0, 1), (0, 2)),
                                constant_values=1.0), f32)
    normP = jnp.asarray(np.pad(_SBF_NORM, ((0, 1), (0, 2))), f32)
    tri_in = [ang2, dkj, tm, g, rootsP, normP, ib['W_sbf1'], ib['W_sbf2']]
    tri_out = pl.pallas_call(
        _tri_body,
        grid=(gt,),
        in_specs=[ts(1), ts(1), ts(1), ts(DOWN), _full(rootsP), _full(normP),
                  _full(ib['W_sbf1']), _full(ib['W_sbf2'])],
        out_specs=ts(ib['W_sbf2'].shape[1]),
        out_shape=jax.ShapeDtypeStruct((Nt, ib['W_sbf2'].shape[1]), f32),
    )(*tri_in)

    seg = jax.ops.segment_sum(tri_out, reduce_to_ji, num_segments=E)

    (rbW1, rbb1, rbW2, rbb2), = ib['res_before']
    (raW1a, rab1a, raW2a, rab2a), (raW1b, rab1b, raW2b, rab2b) = ib['res_after']
    edge2_in = [seg, x_ji, m, rbf, ib['W_up'],
                rbW1, r1(rbb1), rbW2, r1(rbb2),
                ib['W_skip'], r1(ib['b_skip']),
                raW1a, r1(rab1a), raW2a, r1(rab2a),
                raW1b, r1(rab1b), raW2b, r1(rab2b),
                ob1['W_rbf']]
    t1 = pl.pallas_call(
        _edge2_body,
        grid=(ge,),
        in_specs=[es(DOWN), es(EMB), es(EMB), es(NUM_RBF)]
        + [_full(a) for a in edge2_in[4:]],
        out_specs=es(EMB),
        out_shape=jax.ShapeDtypeStruct((E, EMB), f32),
    )(*edge2_in)

    AB = 1000
    ga = N // AB
    import functools
    pa0, pa1 = pl.pallas_call(
        functools.partial(_scatter_body, ablock=AB),
        grid=(ga, ge),
        in_specs=[pl.BlockSpec((Te, 1), lambda a, e: (e, 0)),
                  pl.BlockSpec((Te, EMB), lambda a, e: (e, 0)),
                  pl.BlockSpec((Te, EMB), lambda a, e: (e, 0))],
        out_specs=[pl.BlockSpec((AB, EMB), lambda a, e: (a, 0)),
                   pl.BlockSpec((AB, EMB), lambda a, e: (a, 0))],
        out_shape=[jax.ShapeDtypeStruct((N, EMB), f32),
                   jax.ShapeDtypeStruct((N, EMB), f32)],
    )(idx_i.reshape(E, 1), t0, t1)

    (d01W, d01b), (d02W, d02b), (d03W, d03b) = ob0['dense']
    (d11W, d11b), (d12W, d12b), (d13W, d13b) = ob1['dense']
    atom_in = [pa0, pa1,
               ob0['W_up'], d01W, r1(d01b), d02W, r1(d02b), d03W, r1(d03b),
               ob0['W_final'],
               ob1['W_up'], d11W, r1(d11b), d12W, r1(d12b), d13W, r1(d13b),
               ob1['W_final']]
    as_ = lambda w: pl.BlockSpec((AB, w), lambda a: (a, 0))
    out = pl.pallas_call(
        _atom_body,
        grid=(ga,),
        in_specs=[as_(EMB), as_(EMB)] + [_full(a) for a in atom_in[2:]],
        out_specs=as_(ob0['W_final'].shape[1]),
        out_shape=jax.ShapeDtypeStruct((N, ob0['W_final'].shape[1]), f32),
    )(*atom_in)
    return out
